# baseline (speedup 1.0000x reference)
import functools

import jax
import jax.numpy as jnp
from jax import lax
from jax.experimental import pallas as pl
from jax.experimental.pallas import tpu as pltpu
from jax.experimental.pallas import tpu_sc as plsc

_V = 1000000
_HIST = 50
_BATCH = 4096
_D = 64
_NC = 2
_NS = 16
_NW = _NC * _NS
_BLK = _BATCH // _NW
_SCALE = 8.0

_FULL_COLS = _V // 128
_LAST_COL_V = _FULL_COLS * 128
_TCOL_ITERS = (_FULL_COLS + _NW - 1) // _NW + 1

_mesh = plsc.VectorSubcoreMesh(
    core_axis_name="c", subcore_axis_name="s", num_cores=_NC, num_subcores=_NS
)
_sc_params = pltpu.CompilerParams(
    use_tc_tiling_on_sc=True, needs_layout_passes=False
)


@functools.partial(
    pl.kernel,
    out_type=jax.ShapeDtypeStruct((_V // 2, 2 * _D), jnp.float32),
    mesh=_mesh,
    scratch_types=[
        pltpu.VMEM((_D, 128), jnp.float32),
        pltpu.VMEM((_D, 128), jnp.float32),
        pltpu.VMEM((_D, 2 * _D), jnp.float32),
        pltpu.VMEM((_D, 2 * _D), jnp.float32),
        pltpu.SemaphoreType.DMA,
        pltpu.SemaphoreType.DMA,
        pltpu.SemaphoreType.DMA,
        pltpu.SemaphoreType.DMA,
        pltpu.VMEM((_D, _D), jnp.float32),
        pltpu.VMEM((16, 16), jnp.int32),
    ],
    compiler_params=_sc_params,
)
def _detile(tt_hbm, t2_hbm, in0, in1, ot0, ot1, gsem0, gsem1, osem0, osem1,
            tail_in, pat):
    wid = lax.axis_index("s") * _NC + lax.axis_index("c")
    plane = lax.iota(jnp.int32, 16)

    def fill_pat(d, carry):
        pat[d, pl.ds(0, 16)] = (plane + d) & 15
        return carry

    lax.fori_loop(0, 16, fill_pat, 0)
    ins = (in0, in1)
    ots = (ot0, ot1)
    gsems = (gsem0, gsem1)
    osems = (osem0, osem1)

    def col_of(t):
        return wid + _NW * t

    def issue_in(b, c):
        pltpu.async_copy(
            tt_hbm.at[pl.ds(0, _D), pl.ds(c * 128, 128)], ins[b], gsems[b]
        )

    def transpose(b):
        src = ins[b]
        dst = ots[b]
        lane = lax.iota(jnp.int32, 16)

        def k_body(k, carry):
            vi = lane + 16 * k
            row = lax.shift_right_logical(vi, 1)
            colb = (vi & 1) * _D

            def blk(f0, carry2):
                def d_body(d, carry3):
                    fx = f0 + pat[d, pl.ds(0, 16)]
                    v = plsc.load_gather(src, [fx, vi])
                    plsc.store_scatter(dst, [row, colb + fx], v * _SCALE)
                    return carry3

                lax.fori_loop(0, 16, d_body, 0, unroll=8)
                return carry2

            lax.fori_loop(0, 4, lambda i, c: blk(16 * i, c), 0)
            return carry

        lax.fori_loop(0, 8, k_body, 0)

    def wait_in(b):
        pltpu.make_async_copy(
            tt_hbm.at[pl.ds(0, _D), pl.ds(0, 128)], ins[b], gsems[b]
        ).wait()

    def wait_ot(b):
        pltpu.make_async_copy(
            ots[b], t2_hbm.at[pl.ds(0, _D)], osems[b]
        ).wait()

    issue_in(0, col_of(0))

    def body(tt, carry):
        for b in (0, 1):
            t = 2 * tt + b
            c = col_of(t)
            cn = col_of(t + 1)

            @pl.when(cn < _FULL_COLS)
            def _():
                issue_in(1 - b, cn)

            @pl.when(c < _FULL_COLS)
            def _():
                wait_in(b)

                @pl.when(t >= 2)
                def _():
                    wait_ot(b)

                transpose(b)
                pltpu.async_copy(
                    ots[b], t2_hbm.at[pl.ds(c * _D, _D)], osems[b]
                )
        return carry

    lax.fori_loop(0, (_TCOL_ITERS + 1) // 2, body, 0)
    wait_ot(0)
    wait_ot(1)

    @pl.when(wid == 4)
    def _():
        pltpu.sync_copy(
            tt_hbm.at[pl.ds(0, _D), pl.ds(_LAST_COL_V, _D)],
            tail_in,
        )
        lane = lax.iota(jnp.int32, 16)

        def k_body(k, carry):
            vi = lane + 16 * k
            row = lax.shift_right_logical(vi, 1)
            colb = (vi & 1) * _D

            def d_body(d, carry2):
                fq = (lane + d) & 15

                def blk(i, carry3):
                    fx = 16 * i + fq
                    v = plsc.load_gather(tail_in, [fx, vi])
                    plsc.store_scatter(ot0, [row, colb + fx], v * _SCALE)
                    return carry3

                lax.fori_loop(0, 4, blk, 0, unroll=4)
                return carry2

            lax.fori_loop(0, 16, d_body, 0)
            return carry

        lax.fori_loop(0, 4, k_body, 0)
        pltpu.sync_copy(
            ot0.at[pl.ds(0, 32)],
            t2_hbm.at[pl.ds(_LAST_COL_V // 2, 32)],
        )


@functools.partial(
    pl.kernel,
    out_type=jax.ShapeDtypeStruct((_HIST, _D // 8, _NW, 8, _BLK), jnp.float32),
    mesh=_mesh,
    scratch_types=[
        pltpu.VMEM((_BLK,), jnp.int32),
        pltpu.VMEM((_BLK,), jnp.int32),
        pltpu.VMEM((_BLK,), jnp.int32),
        pltpu.VMEM((_BLK,), jnp.int32),
        pltpu.VMEM((_BLK, 2 * _D), jnp.float32),
        pltpu.VMEM((_BLK, 2 * _D), jnp.float32),
        pltpu.VMEM((_D, _BLK), jnp.float32),
        pltpu.VMEM((_D, _BLK), jnp.float32),
        pltpu.SemaphoreType.DMA,
        pltpu.SemaphoreType.DMA,
        pltpu.SemaphoreType.DMA,
        pltpu.SemaphoreType.DMA,
        pltpu.VMEM((16, 16), jnp.int32),
    ],
    compiler_params=_sc_params,
)
def _emb_lookup(tok_hbm, table_hbm, out_hbm,
                tokr0, tokr1, idx0, idx1, rows0, rows1, stg0, stg1,
                gsem0, gsem1, ssem0, ssem1, pat):
    wid = lax.axis_index("s") * _NC + lax.axis_index("c")
    plane = lax.iota(jnp.int32, 16)

    def fill_pat(d, carry):
        pat[d, pl.ds(0, 16)] = (plane + d) & 15
        return carry

    lax.fori_loop(0, 16, fill_pat, 0)
    tokrs = (tokr0, tokr1)
    idxs = (idx0, idx1)
    rows = (rows0, rows1)
    stgs = (stg0, stg1)
    gsems = (gsem0, gsem1)
    ssems = (ssem0, ssem1)

    def prep_and_issue(b, h):
        pltpu.sync_copy(tok_hbm.at[h, wid], tokrs[b])

        def halve(k, carry):
            sl = pl.ds(16 * k, 16)
            idxs[b][sl] = lax.shift_right_logical(tokrs[b][sl], 1)
            return carry

        lax.fori_loop(0, _BLK // 16, halve, 0)
        pltpu.async_copy(table_hbm.at[idxs[b]], rows[b], gsems[b])

    def transform(b):
        buf = rows[b]
        stg = stgs[b]
        lane = lax.iota(jnp.int32, 16)

        def grp_body(k, carry):
            sl = pl.ds(16 * k, 16)
            row_idx = lane + 16 * k
            par = (tokrs[b][sl] & 1) * _D

            def blk(f0, carry2):
                def d_body(d, carry3):
                    fx = f0 + pat[d, pl.ds(0, 16)]
                    v = plsc.load_gather(buf, [row_idx, par + fx])
                    plsc.store_scatter(stg, [fx, row_idx], v)
                    return carry3

                lax.fori_loop(0, 16, d_body, 0, unroll=8)
                return carry2

            lax.fori_loop(0, 4, lambda i, c: blk(16 * i, c), 0)
            return carry

        lax.fori_loop(0, _BLK // 16, grp_body, 0)

    def wait_gather(b):
        pltpu.make_async_copy(table_hbm.at[idxs[b]], rows[b], gsems[b]).wait()

    def wait_out(b, h):
        for g in range(8):
            pltpu.make_async_copy(
                stgs[b].at[pl.ds(g * 8, 8)], out_hbm.at[h, g, wid], ssems[b]
            ).wait()

    def unit(b, h, g_iter):
        @pl.when(h + 1 < _HIST)
        def _():
            prep_and_issue(1 - b, h + 1)

        wait_gather(b)

        @pl.when(g_iter > 0)
        def _():
            wait_out(b, h)

        transform(b)
        for g in range(8):
            pltpu.async_copy(
                stgs[b].at[pl.ds(g * 8, 8)], out_hbm.at[h, g, wid], ssems[b]
            )

    prep_and_issue(0, 0)

    def body(g_iter, carry):
        unit(0, 2 * g_iter, g_iter)
        unit(1, 2 * g_iter + 1, g_iter)
        return carry

    lax.fori_loop(0, _HIST // 2, body, 0)

    wait_out(0, 0)
    wait_out(1, 0)


def kernel(b_tokens, table):
    tok3 = b_tokens.T.reshape(_HIST, _NW, _BLK).astype(jnp.int32)
    table2 = _detile(table.T)
    out5 = _emb_lookup(tok3, table2)
    return out5.transpose(2, 4, 0, 1, 3).reshape(_BATCH, _HIST, _D)


# speedup vs baseline: 2.2905x; 2.2905x over previous
import functools

import jax
import jax.numpy as jnp
from jax import lax
from jax.experimental import pallas as pl
from jax.experimental.pallas import tpu as pltpu
from jax.experimental.pallas import tpu_sc as plsc

_B = 4096 * 50
_D = 64
_NC = 2
_NS = 16
_NW = _NC * _NS
_BPW = _B // _NW
_CHUNK = 128
_NCH = _BPW // _CHUNK
_SCALE = 8.0

_mesh = plsc.VectorSubcoreMesh(
    core_axis_name="c", subcore_axis_name="s", num_cores=_NC, num_subcores=_NS
)


@functools.partial(
    pl.kernel,
    out_type=jax.ShapeDtypeStruct((_B, _D), jnp.float32),
    mesh=_mesh,
    scratch_types=[
        pltpu.VMEM((_NCH, _CHUNK), jnp.int32),
        pltpu.VMEM((_CHUNK, _D), jnp.float32),
        pltpu.VMEM((_CHUNK, _D), jnp.float32),
        pltpu.SemaphoreType.DMA,
        pltpu.SemaphoreType.DMA,
    ],
    compiler_params=pltpu.CompilerParams(use_tc_tiling_on_sc=False),
)
def _emb_lookup(tokens_hbm, table_hbm, out_hbm, idx_v, rows0, rows1, sem0, sem1):
    wid = lax.axis_index("s") * _NC + lax.axis_index("c")
    base = wid * _BPW
    pltpu.sync_copy(tokens_hbm.at[wid], idx_v)

    bufs = (rows0, rows1)
    sems = (sem0, sem1)

    def issue_gather(c):
        return pltpu.async_copy(
            table_hbm.at[idx_v.at[c]], bufs[c % 2], sems[c % 2]
        )

    handles = [None] * _NCH
    handles[0] = issue_gather(0)
    for c in range(_NCH):
        if c + 1 < _NCH:
            handles[c + 1] = issue_gather(c + 1)
        handles[c].wait()
        buf = bufs[c % 2]

        def _scale_row(r, carry, buf=buf):
            for j in range(_D // 16):
                sl = pl.ds(j * 16, 16)
                buf[r, sl] = buf[r, sl] * _SCALE
            return carry

        lax.fori_loop(0, _CHUNK, _scale_row, 0)

        pltpu.sync_copy(buf, out_hbm.at[pl.ds(base + c * _CHUNK, _CHUNK)])


def kernel(b_tokens, table):
    tokens = b_tokens.reshape(_NW, _NCH, _CHUNK).astype(jnp.int32)
    out = _emb_lookup(tokens, table)
    return out.reshape(b_tokens.shape[0], b_tokens.shape[1], _D)


# speedup vs baseline: 2.3347x; 1.0193x over previous
import functools

import jax
import jax.numpy as jnp
from jax import lax
from jax.experimental import pallas as pl
from jax.experimental.pallas import tpu as pltpu
from jax.experimental.pallas import tpu_sc as plsc

_HIST = 50
_BATCH = 4096
_D = 64
_NC = 2
_NS = 16
_NW = _NC * _NS
_BLK = _BATCH // _NW
_SCALE = 8.0

_mesh = plsc.VectorSubcoreMesh(
    core_axis_name="c", subcore_axis_name="s", num_cores=_NC, num_subcores=_NS
)


@functools.partial(
    pl.kernel,
    out_type=jax.ShapeDtypeStruct((_HIST, _D // 8, _NW, 8, _BLK), jnp.float32),
    mesh=_mesh,
    scratch_types=[
        pltpu.VMEM((_BLK,), jnp.int32),
        pltpu.VMEM((_BLK,), jnp.int32),
        pltpu.VMEM((_BLK, _D), jnp.float32),
        pltpu.VMEM((_BLK, _D), jnp.float32),
        pltpu.VMEM((_D, _BLK), jnp.float32),
        pltpu.VMEM((_D, _BLK), jnp.float32),
        pltpu.SemaphoreType.DMA,
        pltpu.SemaphoreType.DMA,
        pltpu.SemaphoreType.DMA,
        pltpu.SemaphoreType.DMA,
    ],
    compiler_params=pltpu.CompilerParams(
        use_tc_tiling_on_sc=False, needs_layout_passes=False
    ),
)
def _emb_lookup(tok_hbm, table_hbm, out_hbm,
                idx0, idx1, rows0, rows1, stg0, stg1,
                gsem0, gsem1, ssem0, ssem1):
    wid = lax.axis_index("s") * _NC + lax.axis_index("c")
    idxs = (idx0, idx1)
    rows = (rows0, rows1)
    stgs = (stg0, stg1)
    gsems = (gsem0, gsem1)
    ssems = (ssem0, ssem1)

    def prep_and_issue(b, h):
        pltpu.sync_copy(tok_hbm.at[h, wid], idxs[b])
        pltpu.async_copy(table_hbm.at[idxs[b]], rows[b], gsems[b])

    def transform(b):
        buf = rows[b]
        stg = stgs[b]
        lane = lax.iota(jnp.int32, 16)

        def grp_body(k, carry):
            row_idx = lane + 16 * k

            def blk(f0, carry2):
                def d_body(d, carry3):
                    fx = f0 + ((lane + d) & 15)
                    v = plsc.load_gather(buf, [row_idx, fx])
                    plsc.store_scatter(stg, [fx, row_idx], v * _SCALE)
                    return carry3

                lax.fori_loop(0, 16, d_body, 0, unroll=4)
                return carry2

            lax.fori_loop(0, 4, lambda i, c: blk(16 * i, c), 0)
            return carry

        lax.fori_loop(0, _BLK // 16, grp_body, 0)

    def wait_gather(b):
        pltpu.make_async_copy(table_hbm.at[idxs[b]], rows[b], gsems[b]).wait()

    def wait_out(b, h):
        for g in range(8):
            pltpu.make_async_copy(
                stgs[b].at[pl.ds(g * 8, 8)], out_hbm.at[h, g, wid], ssems[b]
            ).wait()

    def unit(b, h, g_iter):
        @pl.when(h + 1 < _HIST)
        def _():
            prep_and_issue(1 - b, h + 1)

        wait_gather(b)

        @pl.when(g_iter > 0)
        def _():
            wait_out(b, h)

        transform(b)
        for g in range(8):
            pltpu.async_copy(
                stgs[b].at[pl.ds(g * 8, 8)], out_hbm.at[h, g, wid], ssems[b]
            )

    prep_and_issue(0, 0)

    def body(g_iter, carry):
        unit(0, 2 * g_iter, g_iter)
        unit(1, 2 * g_iter + 1, g_iter)
        return carry

    lax.fori_loop(0, _HIST // 2, body, 0)

    wait_out(0, 0)
    wait_out(1, 0)


def kernel(b_tokens, table):
    tok3 = b_tokens.T.reshape(_HIST, _NW, _BLK).astype(jnp.int32)
    out5 = _emb_lookup(tok3, table)
    return out5.transpose(2, 4, 0, 1, 3).reshape(_BATCH, _HIST, _D)
